# Initial kernel scaffold; baseline (speedup 1.0000x reference)
import jax
import jax.numpy as jnp
from jax.experimental import pallas as pl


def kernel(x, edge_index, W_self0, W_neigh0, b0, W_self1, W_neigh1, b1, W_self2, W_neigh2, b2):
    raise NotImplementedError("write your pallas kernel here")


# speedup vs baseline: 6.4874x; 6.4874x over previous
import functools

import jax
import jax.numpy as jnp
from jax import lax
from jax.experimental import pallas as pl
from jax.experimental.pallas import tpu as pltpu
from jax.experimental.pallas import tpu_sc as plsc

N = 10000
E = 320000
NC = 2
NS = 16
B = 128
NB = 158
TILE_E = NB * B
E_PAD = NS * TILE_E
DUMMY = N
ACC_ROWS = N + 8
RPT = N // NS
DEG_W = 16


def _seg_sum(table, src_t, dst_t, with_deg):
    W = table.shape[1]
    Wc = W // 2
    th0 = table[:, :Wc]
    th1 = table[:, Wc:]
    zrows = jnp.zeros((RPT, Wc), jnp.float32)

    out_type = [jax.ShapeDtypeStruct((N, W), jnp.float32)]
    scratch = [
        pltpu.VMEM((NB, B), jnp.int32),
        pltpu.VMEM((NB, B), jnp.int32),
        pltpu.VMEM((B, Wc), jnp.float32),
        pltpu.VMEM((B, Wc), jnp.float32),
        pltpu.VMEM_SHARED((ACC_ROWS, Wc), jnp.float32),
        pltpu.SemaphoreType.DMA,
        pltpu.SemaphoreType.DMA,
    ]
    inputs = [th0, th1, src_t, dst_t, zrows]
    if with_deg:
        out_type.append(jax.ShapeDtypeStruct((N, DEG_W), jnp.float32))
        scratch += [
            pltpu.VMEM((B, DEG_W), jnp.float32),
            pltpu.VMEM_SHARED((ACC_ROWS, DEG_W), jnp.float32),
        ]
        inputs += [jnp.ones((B, DEG_W), jnp.float32),
                   jnp.zeros((RPT, DEG_W), jnp.float32)]

    mesh = plsc.VectorSubcoreMesh(core_axis_name="c", subcore_axis_name="s")

    def body(*refs):
        if with_deg:
            (th0_h, th1_h, src_h, dst_h, zr_h, ones_h, zd_h,
             out_h, deg_h,
             srcv, dstv, gb0, gb1, acc, sem0, sem1, onesv, dacc) = refs
        else:
            (th0_h, th1_h, src_h, dst_h, zr_h,
             out_h,
             srcv, dstv, gb0, gb1, acc, sem0, sem1) = refs
        c = lax.axis_index("c")
        s = lax.axis_index("s")

        def run(th, col0, do_deg):
            pltpu.sync_copy(src_h.at[s], srcv)
            pltpu.sync_copy(dst_h.at[s], dstv)
            pltpu.sync_copy(zr_h, acc.at[pl.ds(s * RPT, RPT)])
            if do_deg:
                pltpu.sync_copy(zd_h, dacc.at[pl.ds(s * RPT, RPT)])
                pltpu.sync_copy(ones_h, onesv)
            plsc.subcore_barrier()

            pltpu.async_copy(th.at[srcv.at[0]], gb0, sem0)

            @pl.loop(0, NB, step=2)
            def _(j):
                pltpu.async_copy(th.at[srcv.at[j + 1]], gb1, sem1)
                pltpu.make_async_copy(th.at[srcv.at[j]], gb0, sem0).wait()
                pltpu.sync_copy(gb0, acc.at[dstv.at[j]], add=True)
                if do_deg:
                    pltpu.sync_copy(onesv, dacc.at[dstv.at[j]], add=True)

                @pl.when(j + 2 < NB)
                def _():
                    pltpu.async_copy(th.at[srcv.at[j + 2]], gb0, sem0)

                pltpu.make_async_copy(th.at[srcv.at[j + 1]], gb1, sem1).wait()
                pltpu.sync_copy(gb1, acc.at[dstv.at[j + 1]], add=True)
                if do_deg:
                    pltpu.sync_copy(onesv, dacc.at[dstv.at[j + 1]], add=True)

            plsc.subcore_barrier()
            pltpu.sync_copy(
                acc.at[pl.ds(s * RPT, RPT)],
                out_h.at[pl.ds(s * RPT, RPT), pl.ds(col0, Wc)])
            if do_deg:
                pltpu.sync_copy(dacc.at[pl.ds(s * RPT, RPT)],
                                deg_h.at[pl.ds(s * RPT, RPT)])

        @pl.when(c == 0)
        def _():
            run(th0_h, 0, with_deg)

        @pl.when(c == 1)
        def _():
            run(th1_h, Wc, False)

    f = pl.kernel(body, out_type=tuple(out_type), mesh=mesh,
                  scratch_types=tuple(scratch),
                  compiler_params=pltpu.CompilerParams(
                      use_tc_tiling_on_sc=False))
    out = f(*inputs)
    return out if with_deg else out[0]


_R = 1000


def _tc_self(h, Ws, b):
    di, do = Ws.shape

    def body(h_ref, w_ref, b_ref, o_ref):
        o_ref[...] = jnp.dot(h_ref[...], w_ref[...],
                             preferred_element_type=jnp.float32) + b_ref[...]

    return pl.pallas_call(
        body,
        grid=(N // _R,),
        in_specs=[
            pl.BlockSpec((_R, di), lambda i: (i, 0)),
            pl.BlockSpec((di, do), lambda i: (0, 0)),
            pl.BlockSpec((1, do), lambda i: (0, 0)),
        ],
        out_specs=pl.BlockSpec((_R, do), lambda i: (i, 0)),
        out_shape=jax.ShapeDtypeStruct((N, do), jnp.float32),
    )(h, Ws, b.reshape(1, do))


def _tc_post(s, acc, deg, Wn, relu):
    di, do = Wn.shape

    def body(s_ref, a_ref, d_ref, w_ref, o_ref):
        dinv = 1.0 / jnp.maximum(d_ref[...][:, 0:1], 1.0)
        y = s_ref[...] + jnp.dot(a_ref[...] * dinv, w_ref[...],
                                 preferred_element_type=jnp.float32)
        o_ref[...] = jnp.maximum(y, 0.0) if relu else y

    return pl.pallas_call(
        body,
        grid=(N // _R,),
        in_specs=[
            pl.BlockSpec((_R, do), lambda i: (i, 0)),
            pl.BlockSpec((_R, di), lambda i: (i, 0)),
            pl.BlockSpec((_R, DEG_W), lambda i: (i, 0)),
            pl.BlockSpec((di, do), lambda i: (0, 0)),
        ],
        out_specs=pl.BlockSpec((_R, do), lambda i: (i, 0)),
        out_shape=jax.ShapeDtypeStruct((N, do), jnp.float32),
    )(s, acc, deg, Wn)


def _tc_post_fused(s1, acc1, deg, Wn1, Wn2, Ws2, b2):
    di, dh = Wn1.shape
    do = Wn2.shape[1]

    def body(s_ref, a_ref, d_ref, wn1_ref, wn2_ref, ws2_ref, b2_ref,
             z_ref, o_ref):
        dinv = 1.0 / jnp.maximum(d_ref[...][:, 0:1], 1.0)
        h2 = s_ref[...] + jnp.dot(a_ref[...] * dinv, wn1_ref[...],
                                  preferred_element_type=jnp.float32)
        h2 = jnp.maximum(h2, 0.0)
        z_ref[...] = jnp.dot(h2, wn2_ref[...],
                             preferred_element_type=jnp.float32)
        o_ref[...] = jnp.dot(h2, ws2_ref[...],
                             preferred_element_type=jnp.float32) + b2_ref[...]

    return pl.pallas_call(
        body,
        grid=(N // _R,),
        in_specs=[
            pl.BlockSpec((_R, dh), lambda i: (i, 0)),
            pl.BlockSpec((_R, di), lambda i: (i, 0)),
            pl.BlockSpec((_R, DEG_W), lambda i: (i, 0)),
            pl.BlockSpec((di, dh), lambda i: (0, 0)),
            pl.BlockSpec((dh, do), lambda i: (0, 0)),
            pl.BlockSpec((dh, do), lambda i: (0, 0)),
            pl.BlockSpec((1, do), lambda i: (0, 0)),
        ],
        out_specs=[
            pl.BlockSpec((_R, do), lambda i: (i, 0)),
            pl.BlockSpec((_R, do), lambda i: (i, 0)),
        ],
        out_shape=[
            jax.ShapeDtypeStruct((N, do), jnp.float32),
            jax.ShapeDtypeStruct((N, do), jnp.float32),
        ],
    )(s1, acc1, deg, Wn1, Wn2, Ws2, b2.reshape(1, do))


def _tc_final(s2, acc2, deg):
    do = s2.shape[1]

    def body(s_ref, a_ref, d_ref, o_ref):
        dinv = 1.0 / jnp.maximum(d_ref[...][:, 0:1], 1.0)
        o_ref[...] = s_ref[...] + a_ref[...] * dinv

    return pl.pallas_call(
        body,
        grid=(N // _R,),
        in_specs=[
            pl.BlockSpec((_R, do), lambda i: (i, 0)),
            pl.BlockSpec((_R, do), lambda i: (i, 0)),
            pl.BlockSpec((_R, DEG_W), lambda i: (i, 0)),
        ],
        out_specs=pl.BlockSpec((_R, do), lambda i: (i, 0)),
        out_shape=jax.ShapeDtypeStruct((N, do), jnp.float32),
    )(s2, acc2, deg)


def kernel(x, edge_index, W_self0, W_neigh0, b0,
           W_self1, W_neigh1, b1, W_self2, W_neigh2, b2):
    src = edge_index[0]
    dst = edge_index[1]
    pad = E_PAD - E
    src_t = jnp.concatenate(
        [src, jnp.zeros((pad,), jnp.int32)]).reshape(NS, NB, B)
    dst_t = jnp.concatenate(
        [dst, jnp.full((pad,), DUMMY, jnp.int32)]).reshape(NS, NB, B)

    acc0, deg = _seg_sum(x, src_t, dst_t, with_deg=True)
    s0 = _tc_self(x, W_self0, b0)
    h1 = _tc_post(s0, acc0, deg, W_neigh0, relu=True)

    h1_halves = jnp.stack([h1[:, :128], h1[:, 128:]])

    def _scan_step(carry, tab):
        return carry, _seg_sum(tab, src_t, dst_t, with_deg=False)

    _, acc1_halves = lax.scan(_scan_step, 0, h1_halves)
    acc1 = acc1_halves.transpose(1, 0, 2).reshape(N, 256)
    s1 = _tc_self(h1, W_self1, b1)

    z2, s2 = _tc_post_fused(s1, acc1, deg, W_neigh1, W_neigh2, W_self2, b2)
    acc2 = _seg_sum(z2, src_t, dst_t, with_deg=False)
    return _tc_final(s2, acc2, deg)


# speedup vs baseline: 6.6180x; 1.0201x over previous
import functools

import jax
import jax.numpy as jnp
from jax import lax
from jax.experimental import pallas as pl
from jax.experimental.pallas import tpu as pltpu
from jax.experimental.pallas import tpu_sc as plsc

N = 10000
E = 320000
NC = 2
NS = 16
B = 128
NB = 158
NBH = NB // 2
TILE_E = NB * B
E_PAD = NS * TILE_E
DUMMY = N
ACC_ROWS = N + 8
RPT = N // NS
DEG_W = 16


def _seg_sum(table, src_t, dst_t, with_deg):
    W = table.shape[1]
    Wc = W // 2
    th0 = table[:, :Wc]
    th1 = table[:, Wc:]
    zrows = jnp.zeros((RPT, Wc), jnp.float32)

    out_type = [jax.ShapeDtypeStruct((N, W), jnp.float32)]
    scratch = [
        pltpu.VMEM((NB, B), jnp.int32),
        pltpu.VMEM((NB, B), jnp.int32),
        pltpu.VMEM((B, Wc), jnp.float32),
        pltpu.VMEM((B, Wc), jnp.float32),
        pltpu.VMEM_SHARED((ACC_ROWS, Wc), jnp.float32),
        pltpu.SemaphoreType.DMA,
        pltpu.SemaphoreType.DMA,
    ]
    inputs = [th0, th1, src_t, dst_t, zrows]
    if with_deg:
        out_type.append(jax.ShapeDtypeStruct((NC, N, DEG_W), jnp.float32))
        scratch += [
            pltpu.VMEM((B, DEG_W), jnp.float32),
            pltpu.VMEM_SHARED((ACC_ROWS, DEG_W), jnp.float32),
        ]
        inputs += [jnp.ones((B, DEG_W), jnp.float32),
                   jnp.zeros((RPT, DEG_W), jnp.float32)]

    mesh = plsc.VectorSubcoreMesh(core_axis_name="c", subcore_axis_name="s")

    def body(*refs):
        if with_deg:
            (th0_h, th1_h, src_h, dst_h, zr_h, ones_h, zd_h,
             out_h, deg_h,
             srcv, dstv, gb0, gb1, acc, sem0, sem1, onesv, dacc) = refs
        else:
            (th0_h, th1_h, src_h, dst_h, zr_h,
             out_h,
             srcv, dstv, gb0, gb1, acc, sem0, sem1) = refs
        c = lax.axis_index("c")
        s = lax.axis_index("s")

        def run(th, col0, deg_lo, deg_hi):
            pltpu.sync_copy(src_h.at[s], srcv)
            pltpu.sync_copy(dst_h.at[s], dstv)
            pltpu.sync_copy(zr_h, acc.at[pl.ds(s * RPT, RPT)])
            if with_deg:
                pltpu.sync_copy(zd_h, dacc.at[pl.ds(s * RPT, RPT)])
                pltpu.sync_copy(ones_h, onesv)
            plsc.subcore_barrier()

            pltpu.async_copy(th.at[srcv.at[0]], gb0, sem0)

            @pl.loop(0, NB, step=2)
            def _(j):
                pltpu.async_copy(th.at[srcv.at[j + 1]], gb1, sem1)
                pltpu.make_async_copy(th.at[srcv.at[j]], gb0, sem0).wait()
                pltpu.sync_copy(gb0, acc.at[dstv.at[j]], add=True)
                if with_deg:
                    @pl.when(jnp.logical_and(j >= deg_lo, j < deg_hi))
                    def _():
                        pltpu.sync_copy(onesv, dacc.at[dstv.at[j]], add=True)

                @pl.when(j + 2 < NB)
                def _():
                    pltpu.async_copy(th.at[srcv.at[j + 2]], gb0, sem0)

                pltpu.make_async_copy(th.at[srcv.at[j + 1]], gb1, sem1).wait()
                pltpu.sync_copy(gb1, acc.at[dstv.at[j + 1]], add=True)
                if with_deg:
                    @pl.when(jnp.logical_and(j + 1 >= deg_lo, j + 1 < deg_hi))
                    def _():
                        pltpu.sync_copy(onesv, dacc.at[dstv.at[j + 1]], add=True)

            plsc.subcore_barrier()
            pltpu.sync_copy(
                acc.at[pl.ds(s * RPT, RPT)],
                out_h.at[pl.ds(s * RPT, RPT), pl.ds(col0, Wc)])
            if with_deg:
                cidx = 0 if deg_lo == 0 else 1
                pltpu.sync_copy(dacc.at[pl.ds(s * RPT, RPT)],
                                deg_h.at[cidx, pl.ds(s * RPT, RPT)])

        @pl.when(c == 0)
        def _():
            run(th0_h, 0, 0, NBH)

        @pl.when(c == 1)
        def _():
            run(th1_h, Wc, NBH, NB)

    f = pl.kernel(body, out_type=tuple(out_type), mesh=mesh,
                  scratch_types=tuple(scratch),
                  compiler_params=pltpu.CompilerParams(
                      use_tc_tiling_on_sc=False))
    out = f(*inputs)
    return out if with_deg else out[0]


_R = 1000


def _deg_inv(d_ref):
    d = d_ref[0][...][0, :, 0:1] + d_ref[1][...][0, :, 0:1]
    return 1.0 / jnp.maximum(d, 1.0)


def _tc_self(h, Ws, b):
    di, do = Ws.shape

    def body(h_ref, w_ref, b_ref, o_ref):
        o_ref[...] = jnp.dot(h_ref[...], w_ref[...],
                             preferred_element_type=jnp.float32) + b_ref[...]

    return pl.pallas_call(
        body,
        grid=(N // _R,),
        in_specs=[
            pl.BlockSpec((_R, di), lambda i: (i, 0)),
            pl.BlockSpec((di, do), lambda i: (0, 0)),
            pl.BlockSpec((1, do), lambda i: (0, 0)),
        ],
        out_specs=pl.BlockSpec((_R, do), lambda i: (i, 0)),
        out_shape=jax.ShapeDtypeStruct((N, do), jnp.float32),
    )(h, Ws, b.reshape(1, do))


def _tc_layer1(s0, acc0, deg, Wn0, Ws1, b1):
    di, dh = Wn0.shape

    def body(s_ref, a_ref, d0_ref, d1_ref, wn_ref, ws_ref, b_ref,
             hh_ref, s1_ref):
        dinv = _deg_inv((d0_ref, d1_ref))
        h1 = s_ref[...] + jnp.dot(a_ref[...] * dinv, wn_ref[...],
                                  preferred_element_type=jnp.float32)
        h1 = jnp.maximum(h1, 0.0)
        hh_ref[0] = h1[:, :dh // 2]
        hh_ref[1] = h1[:, dh // 2:]
        s1_ref[...] = jnp.dot(h1, ws_ref[...],
                              preferred_element_type=jnp.float32) + b_ref[...]

    return pl.pallas_call(
        body,
        grid=(N // _R,),
        in_specs=[
            pl.BlockSpec((_R, dh), lambda i: (i, 0)),
            pl.BlockSpec((_R, di), lambda i: (i, 0)),
            pl.BlockSpec((1, _R, DEG_W), lambda i: (0, i, 0)),
            pl.BlockSpec((1, _R, DEG_W), lambda i: (1, i, 0)),
            pl.BlockSpec((di, dh), lambda i: (0, 0)),
            pl.BlockSpec((dh, dh), lambda i: (0, 0)),
            pl.BlockSpec((1, dh), lambda i: (0, 0)),
        ],
        out_specs=[
            pl.BlockSpec((2, _R, dh // 2), lambda i: (0, i, 0)),
            pl.BlockSpec((_R, dh), lambda i: (i, 0)),
        ],
        out_shape=[
            jax.ShapeDtypeStruct((2, N, dh // 2), jnp.float32),
            jax.ShapeDtypeStruct((N, dh), jnp.float32),
        ],
    )(s0, acc0, deg, deg, Wn0, Ws1, b1.reshape(1, dh))


def _tc_layer2(s1, acc1h, deg, Wn1, Wn2, Ws2, b2):
    dh = Wn1.shape[0]
    do = Wn2.shape[1]

    def body(s_ref, a_ref, d0_ref, d1_ref, wn1_ref, wn2_ref, ws2_ref, b2_ref,
             z_ref, o_ref):
        dinv = _deg_inv((d0_ref, d1_ref))
        agg = jnp.concatenate([a_ref[0], a_ref[1]], axis=1) * dinv
        h2 = s_ref[...] + jnp.dot(agg, wn1_ref[...],
                                  preferred_element_type=jnp.float32)
        h2 = jnp.maximum(h2, 0.0)
        z_ref[...] = jnp.dot(h2, wn2_ref[...],
                             preferred_element_type=jnp.float32)
        o_ref[...] = jnp.dot(h2, ws2_ref[...],
                             preferred_element_type=jnp.float32) + b2_ref[...]

    return pl.pallas_call(
        body,
        grid=(N // _R,),
        in_specs=[
            pl.BlockSpec((_R, dh), lambda i: (i, 0)),
            pl.BlockSpec((2, _R, dh // 2), lambda i: (0, i, 0)),
            pl.BlockSpec((1, _R, DEG_W), lambda i: (0, i, 0)),
            pl.BlockSpec((1, _R, DEG_W), lambda i: (1, i, 0)),
            pl.BlockSpec((dh, dh), lambda i: (0, 0)),
            pl.BlockSpec((dh, do), lambda i: (0, 0)),
            pl.BlockSpec((dh, do), lambda i: (0, 0)),
            pl.BlockSpec((1, do), lambda i: (0, 0)),
        ],
        out_specs=[
            pl.BlockSpec((_R, do), lambda i: (i, 0)),
            pl.BlockSpec((_R, do), lambda i: (i, 0)),
        ],
        out_shape=[
            jax.ShapeDtypeStruct((N, do), jnp.float32),
            jax.ShapeDtypeStruct((N, do), jnp.float32),
        ],
    )(s1, acc1h, deg, deg, Wn1, Wn2, Ws2, b2.reshape(1, do))


def _tc_final(s2, acc2, deg):
    do = s2.shape[1]

    def body(s_ref, a_ref, d0_ref, d1_ref, o_ref):
        dinv = _deg_inv((d0_ref, d1_ref))
        o_ref[...] = s_ref[...] + a_ref[...] * dinv

    return pl.pallas_call(
        body,
        grid=(N // _R,),
        in_specs=[
            pl.BlockSpec((_R, do), lambda i: (i, 0)),
            pl.BlockSpec((_R, do), lambda i: (i, 0)),
            pl.BlockSpec((1, _R, DEG_W), lambda i: (0, i, 0)),
            pl.BlockSpec((1, _R, DEG_W), lambda i: (1, i, 0)),
        ],
        out_specs=pl.BlockSpec((_R, do), lambda i: (i, 0)),
        out_shape=jax.ShapeDtypeStruct((N, do), jnp.float32),
    )(s2, acc2, deg, deg)


def kernel(x, edge_index, W_self0, W_neigh0, b0,
           W_self1, W_neigh1, b1, W_self2, W_neigh2, b2):
    src = edge_index[0]
    dst = edge_index[1]
    pad = E_PAD - E
    src_t = jnp.concatenate(
        [src, jnp.zeros((pad,), jnp.int32)]).reshape(NS, NB, B)
    dst_t = jnp.concatenate(
        [dst, jnp.full((pad,), DUMMY, jnp.int32)]).reshape(NS, NB, B)

    acc0, deg = _seg_sum(x, src_t, dst_t, with_deg=True)
    s0 = _tc_self(x, W_self0, b0)
    h1h, s1 = _tc_layer1(s0, acc0, deg, W_neigh0, W_self1, b1)

    def _scan_step(carry, tab):
        return carry, _seg_sum(tab, src_t, dst_t, with_deg=False)

    _, acc1h = lax.scan(_scan_step, 0, h1h)

    z2, s2 = _tc_layer2(s1, acc1h, deg, W_neigh1, W_neigh2, W_self2, b2)
    acc2 = _seg_sum(z2, src_t, dst_t, with_deg=False)
    return _tc_final(s2, acc2, deg)
